# Initial kernel scaffold; baseline (speedup 1.0000x reference)
import jax
import jax.numpy as jnp
from jax.experimental import pallas as pl


def kernel(x, edge_index, W1, b1, W2, b2):
    raise NotImplementedError("write your pallas kernel here")


# speedup vs baseline: 25.9537x; 25.9537x over previous
import functools

import jax
import jax.numpy as jnp
from jax import lax
from jax.experimental import pallas as pl
from jax.experimental.pallas import tpu as pltpu
from jax.experimental.pallas import tpu_sc as plsc

NUM_NODES = 100000
NUM_EDGES = 1600000
NC = 2
NS = 16
NW = NC * NS
CHUNK = 128
STAGE = 8
TILE_CHUNKS = 392
OUTER = TILE_CHUNKS // STAGE
TOTAL_CHUNKS = TILE_CHUNKS * NW
EDGES_PAD = TOTAL_CHUNKS * CHUNK
NODES_PAD = 100352
SLICE = NODES_PAD // NS


@functools.lru_cache(maxsize=None)
def _make_sc_pass(feat, do_gather):
  mesh = plsc.VectorSubcoreMesh(core_axis_name="c", subcore_axis_name="s",
                                num_cores=NC, num_subcores=NS)
  scratch = []
  if do_gather:
    scratch.append(pltpu.VMEM((STAGE, CHUNK), jnp.int32))
  scratch += [
      pltpu.VMEM((STAGE, CHUNK), jnp.int32),
      pltpu.VMEM((CHUNK, feat), jnp.float32),
      pltpu.VMEM_SHARED((NODES_PAD, feat), jnp.float32),
      pltpu.SemaphoreType.DMA,
  ]

  def body(*refs):
    if do_gather:
      (table, src2d, dst2d, zeros_blk, out,
       src_v, dst_v, rows_v, acc, sem) = refs
    else:
      (table, dst2d, zeros_blk, out,
       dst_v, rows_v, acc, sem) = refs
      src2d = src_v = None
    cid = lax.axis_index("c")
    sid = lax.axis_index("s")
    wid = sid * NC + cid
    pltpu.sync_copy(zeros_blk, acc.at[pl.ds(sid * SLICE, SLICE)])
    if not do_gather:
      pltpu.sync_copy(table, rows_v)
    plsc.subcore_barrier()

    row0 = wid * TILE_CHUNKS

    def outer(o, carry):
      base = row0 + o * STAGE
      pltpu.sync_copy(dst2d.at[pl.ds(base, STAGE)], dst_v)
      if do_gather:
        pltpu.sync_copy(src2d.at[pl.ds(base, STAGE)], src_v)
      for j in range(STAGE):
        if do_gather:
          pltpu.async_copy(table.at[src_v.at[j]], rows_v, sem).wait()
        pltpu.sync_copy(rows_v, acc.at[dst_v.at[j]], add=True)
      return carry

    lax.fori_loop(0, OUTER, outer, 0)
    plsc.subcore_barrier()
    pltpu.sync_copy(acc.at[pl.ds(sid * SLICE, SLICE)],
                    out.at[cid, pl.ds(sid * SLICE, SLICE)])

  return functools.partial(
      pl.kernel,
      out_type=jax.ShapeDtypeStruct((NC, NODES_PAD, feat), jnp.float32),
      mesh=mesh,
      scratch_types=scratch,
      compiler_params=pltpu.CompilerParams(use_tc_tiling_on_sc=False),
  )(body)


BLK = 2048
GRID = NODES_PAD // BLK


def _tc_prep_body(degp_ref, x_ref, g1_ref, dis_ref):
  deg = degp_ref[0] + degp_ref[1] + 1.0
  dis = lax.rsqrt(deg)
  dis_ref[...] = dis
  g1_ref[...] = x_ref[...] * dis


_tc_prep = pl.pallas_call(
    _tc_prep_body,
    grid=(GRID,),
    in_specs=[
        pl.BlockSpec((NC, BLK, 1), lambda i: (0, i, 0)),
        pl.BlockSpec((BLK, 4), lambda i: (i, 0)),
    ],
    out_specs=[
        pl.BlockSpec((BLK, 4), lambda i: (i, 0)),
        pl.BlockSpec((BLK, 1), lambda i: (i, 0)),
    ],
    out_shape=[
        jax.ShapeDtypeStruct((NODES_PAD, 4), jnp.float32),
        jax.ShapeDtypeStruct((NODES_PAD, 1), jnp.float32),
    ],
)


def _tc_mid_body(a1p_ref, x_ref, dis_ref, w1_ref, b1_ref, w2_ref,
                 g2_ref, sp2_ref):
  dis = dis_ref[...]
  dis2 = dis * dis
  a1 = a1p_ref[0] + a1p_ref[1]
  z = dis * a1 + dis2 * x_ref[...]
  h = jnp.dot(z, w1_ref[...], preferred_element_type=jnp.float32)
  h = jnp.maximum(h + b1_ref[...], 0.0)
  p = jnp.dot(h, w2_ref[...], preferred_element_type=jnp.float32)
  g2_ref[...] = dis * p
  sp2_ref[...] = dis2 * p


_tc_mid = pl.pallas_call(
    _tc_mid_body,
    grid=(GRID,),
    in_specs=[
        pl.BlockSpec((NC, BLK, 4), lambda i: (0, i, 0)),
        pl.BlockSpec((BLK, 4), lambda i: (i, 0)),
        pl.BlockSpec((BLK, 1), lambda i: (i, 0)),
        pl.BlockSpec((4, 64), lambda i: (0, 0)),
        pl.BlockSpec((1, 64), lambda i: (0, 0)),
        pl.BlockSpec((64, 2), lambda i: (0, 0)),
    ],
    out_specs=[
        pl.BlockSpec((BLK, 2), lambda i: (i, 0)),
        pl.BlockSpec((BLK, 2), lambda i: (i, 0)),
    ],
    out_shape=[
        jax.ShapeDtypeStruct((NODES_PAD, 2), jnp.float32),
        jax.ShapeDtypeStruct((NODES_PAD, 2), jnp.float32),
    ],
)


def _tc_final_body(a2p_ref, sp2_ref, dis_ref, b2_ref, out_ref):
  out_ref[...] = (dis_ref[...] * (a2p_ref[0] + a2p_ref[1])
                  + sp2_ref[...] + b2_ref[...])


_tc_final = pl.pallas_call(
    _tc_final_body,
    grid=(GRID,),
    in_specs=[
        pl.BlockSpec((NC, BLK, 2), lambda i: (0, i, 0)),
        pl.BlockSpec((BLK, 2), lambda i: (i, 0)),
        pl.BlockSpec((BLK, 1), lambda i: (i, 0)),
        pl.BlockSpec((1, 2), lambda i: (0, 0)),
    ],
    out_specs=pl.BlockSpec((BLK, 2), lambda i: (i, 0)),
    out_shape=jax.ShapeDtypeStruct((NODES_PAD, 2), jnp.float32),
)


def kernel(x, edge_index, W1, b1, W2, b2):
  src = edge_index[0]
  dst = edge_index[1]
  pad = EDGES_PAD - NUM_EDGES
  ar = jnp.arange(pad, dtype=jnp.int32)
  src2d = jnp.concatenate([src, ar % NUM_NODES]).reshape(TOTAL_CHUNKS, CHUNK)
  dst2d = jnp.concatenate([dst, NUM_NODES + (ar % CHUNK)]).reshape(
      TOTAL_CHUNKS, CHUNK)
  x_pad = jnp.zeros((NODES_PAD, 4), jnp.float32).at[:NUM_NODES].set(x)
  ones_rows = jnp.ones((CHUNK, 1), jnp.float32)
  z1 = jnp.zeros((SLICE, 1), jnp.float32)
  z4 = jnp.zeros((SLICE, 4), jnp.float32)
  z2 = jnp.zeros((SLICE, 2), jnp.float32)

  degp = _make_sc_pass(1, False)(ones_rows, dst2d, z1)
  g1, dis = _tc_prep(degp, x_pad)
  a1p = _make_sc_pass(4, True)(g1, src2d, dst2d, z4)
  g2, sp2 = _tc_mid(a1p, x_pad, dis, W1, b1.reshape(1, 64), W2)
  a2p = _make_sc_pass(2, True)(g2, src2d, dst2d, z2)
  out = _tc_final(a2p, sp2, dis, b2.reshape(1, 2))
  return out[:NUM_NODES]


# speedup vs baseline: 35.8532x; 1.3814x over previous
import functools

import jax
import jax.numpy as jnp
from jax import lax
from jax.experimental import pallas as pl
from jax.experimental.pallas import tpu as pltpu
from jax.experimental.pallas import tpu_sc as plsc

NUM_NODES = 100000
NUM_EDGES = 1600000
NC = 2
NS = 16
NW = NC * NS
CHUNK = 128
STAGE = 8
TILE_CHUNKS = 392
OUTER = TILE_CHUNKS // STAGE
TOTAL_CHUNKS = TILE_CHUNKS * NW
EDGES_PAD = TOTAL_CHUNKS * CHUNK
NODES_PAD = 100352
SLICE = NODES_PAD // NS


@functools.lru_cache(maxsize=None)
def _make_sc_pass(feat, do_gather):
  mesh = plsc.VectorSubcoreMesh(core_axis_name="c", subcore_axis_name="s",
                                num_cores=NC, num_subcores=NS)
  scratch = []
  if do_gather:
    scratch.append(pltpu.VMEM((STAGE, CHUNK), jnp.int32))
  scratch += [
      pltpu.VMEM((STAGE, CHUNK), jnp.int32),
      pltpu.VMEM((STAGE, CHUNK, feat), jnp.float32),
      pltpu.VMEM_SHARED((NODES_PAD, feat), jnp.float32),
      pltpu.SemaphoreType.DMA,
      pltpu.SemaphoreType.DMA,
  ]

  def body(*refs):
    if do_gather:
      (table, src2d, dst2d, zeros_blk, out,
       src_v, dst_v, rows_v, acc, sem_g, sem_s) = refs
    else:
      (table, dst2d, zeros_blk, out,
       dst_v, rows_v, acc, sem_g, sem_s) = refs
      src2d = src_v = None
    cid = lax.axis_index("c")
    sid = lax.axis_index("s")
    wid = sid * NC + cid
    pltpu.sync_copy(zeros_blk, acc.at[pl.ds(sid * SLICE, SLICE)])
    if not do_gather:
      for j in range(STAGE):
        pltpu.sync_copy(table, rows_v.at[j])
    plsc.subcore_barrier()

    row0 = wid * TILE_CHUNKS

    def outer(o, carry):
      base = row0 + o * STAGE
      pltpu.sync_copy(dst2d.at[pl.ds(base, STAGE)], dst_v)
      if do_gather:
        pltpu.sync_copy(src2d.at[pl.ds(base, STAGE)], src_v)
        gd = [pltpu.async_copy(table.at[src_v.at[j]], rows_v.at[j], sem_g)
              for j in range(STAGE)]
      sd = []
      for j in range(STAGE):
        if do_gather:
          gd[j].wait()
        sd.append(pltpu.async_copy(rows_v.at[j], acc.at[dst_v.at[j]], sem_s,
                                   add=True))
      for d in sd:
        d.wait()
      return carry

    lax.fori_loop(0, OUTER, outer, 0)
    plsc.subcore_barrier()
    pltpu.sync_copy(acc.at[pl.ds(sid * SLICE, SLICE)],
                    out.at[cid, pl.ds(sid * SLICE, SLICE)])

  return functools.partial(
      pl.kernel,
      out_type=jax.ShapeDtypeStruct((NC, NODES_PAD, feat), jnp.float32),
      mesh=mesh,
      scratch_types=scratch,
      compiler_params=pltpu.CompilerParams(use_tc_tiling_on_sc=False),
  )(body)


BLK = 2048
GRID = NODES_PAD // BLK


def _tc_prep_body(degp_ref, x_ref, g1_ref, dis_ref):
  deg = degp_ref[0] + degp_ref[1] + 1.0
  dis = lax.rsqrt(deg)
  dis_ref[...] = dis
  g1_ref[...] = x_ref[...] * dis


_tc_prep = pl.pallas_call(
    _tc_prep_body,
    grid=(GRID,),
    in_specs=[
        pl.BlockSpec((NC, BLK, 1), lambda i: (0, i, 0)),
        pl.BlockSpec((BLK, 4), lambda i: (i, 0)),
    ],
    out_specs=[
        pl.BlockSpec((BLK, 4), lambda i: (i, 0)),
        pl.BlockSpec((BLK, 1), lambda i: (i, 0)),
    ],
    out_shape=[
        jax.ShapeDtypeStruct((NODES_PAD, 4), jnp.float32),
        jax.ShapeDtypeStruct((NODES_PAD, 1), jnp.float32),
    ],
)


def _tc_mid_body(a1p_ref, x_ref, dis_ref, w1_ref, b1_ref, w2_ref,
                 g2_ref, sp2_ref):
  dis = dis_ref[...]
  dis2 = dis * dis
  a1 = a1p_ref[0] + a1p_ref[1]
  z = dis * a1 + dis2 * x_ref[...]
  h = jnp.dot(z, w1_ref[...], preferred_element_type=jnp.float32)
  h = jnp.maximum(h + b1_ref[...], 0.0)
  p = jnp.dot(h, w2_ref[...], preferred_element_type=jnp.float32)
  g2_ref[...] = dis * p
  sp2_ref[...] = dis2 * p


_tc_mid = pl.pallas_call(
    _tc_mid_body,
    grid=(GRID,),
    in_specs=[
        pl.BlockSpec((NC, BLK, 4), lambda i: (0, i, 0)),
        pl.BlockSpec((BLK, 4), lambda i: (i, 0)),
        pl.BlockSpec((BLK, 1), lambda i: (i, 0)),
        pl.BlockSpec((4, 64), lambda i: (0, 0)),
        pl.BlockSpec((1, 64), lambda i: (0, 0)),
        pl.BlockSpec((64, 2), lambda i: (0, 0)),
    ],
    out_specs=[
        pl.BlockSpec((BLK, 2), lambda i: (i, 0)),
        pl.BlockSpec((BLK, 2), lambda i: (i, 0)),
    ],
    out_shape=[
        jax.ShapeDtypeStruct((NODES_PAD, 2), jnp.float32),
        jax.ShapeDtypeStruct((NODES_PAD, 2), jnp.float32),
    ],
)


def _tc_final_body(a2p_ref, sp2_ref, dis_ref, b2_ref, out_ref):
  out_ref[...] = (dis_ref[...] * (a2p_ref[0] + a2p_ref[1])
                  + sp2_ref[...] + b2_ref[...])


_tc_final = pl.pallas_call(
    _tc_final_body,
    grid=(GRID,),
    in_specs=[
        pl.BlockSpec((NC, BLK, 2), lambda i: (0, i, 0)),
        pl.BlockSpec((BLK, 2), lambda i: (i, 0)),
        pl.BlockSpec((BLK, 1), lambda i: (i, 0)),
        pl.BlockSpec((1, 2), lambda i: (0, 0)),
    ],
    out_specs=pl.BlockSpec((BLK, 2), lambda i: (i, 0)),
    out_shape=jax.ShapeDtypeStruct((NODES_PAD, 2), jnp.float32),
)


def kernel(x, edge_index, W1, b1, W2, b2):
  src = edge_index[0]
  dst = edge_index[1]
  pad = EDGES_PAD - NUM_EDGES
  ar = jnp.arange(pad, dtype=jnp.int32)
  src2d = jnp.concatenate([src, ar % NUM_NODES]).reshape(TOTAL_CHUNKS, CHUNK)
  dst2d = jnp.concatenate([dst, NUM_NODES + (ar % CHUNK)]).reshape(
      TOTAL_CHUNKS, CHUNK)
  x_pad = jnp.zeros((NODES_PAD, 4), jnp.float32).at[:NUM_NODES].set(x)
  ones_rows = jnp.ones((CHUNK, 1), jnp.float32)
  z1 = jnp.zeros((SLICE, 1), jnp.float32)
  z4 = jnp.zeros((SLICE, 4), jnp.float32)
  z2 = jnp.zeros((SLICE, 2), jnp.float32)

  degp = _make_sc_pass(1, False)(ones_rows, dst2d, z1)
  g1, dis = _tc_prep(degp, x_pad)
  a1p = _make_sc_pass(4, True)(g1, src2d, dst2d, z4)
  g2, sp2 = _tc_mid(a1p, x_pad, dis, W1, b1.reshape(1, 64), W2)
  a2p = _make_sc_pass(2, True)(g2, src2d, dst2d, z2)
  out = _tc_final(a2p, sp2, dis, b2.reshape(1, 2))
  return out[:NUM_NODES]
